# baseline (speedup 1.0000x reference)
import functools

import jax
import jax.numpy as jnp
from jax import lax
from jax.experimental import pallas as pl
from jax.experimental.pallas import tpu as pltpu
from jax.experimental.pallas import tpu_sc as plsc

B = 16384
D = 64
DP = 2 * D
L = 16
NC = 2
NS = 16
NW = NC * NS
BPW = B // NW
CHUNK = 128
NCHUNK = BPW // CHUNK
HALF = 256
NGROUP = HALF // L

_mesh = plsc.VectorSubcoreMesh(core_axis_name="c", subcore_axis_name="s")


@functools.partial(
    pl.kernel,
    mesh=_mesh,
    compiler_params=pltpu.CompilerParams(needs_layout_passes=False),
    out_type=jax.ShapeDtypeStruct((B,), jnp.float32),
    scratch_types=[
        pltpu.VMEM((NCHUNK, CHUNK), jnp.int32),
        pltpu.VMEM((NCHUNK, CHUNK), jnp.int32),
        pltpu.VMEM((NCHUNK, CHUNK), jnp.int32),
        pltpu.VMEM((NCHUNK, CHUNK), jnp.int32),
        pltpu.VMEM((HALF, DP), jnp.float32),
        pltpu.VMEM((HALF, DP), jnp.float32),
        pltpu.VMEM((D,), jnp.float32),
        pltpu.VMEM((L,), jnp.float32),
        pltpu.VMEM((BPW,), jnp.float32),
        pltpu.SemaphoreType.DMA,
        pltpu.SemaphoreType.DMA,
    ],
)
def _sc_kernel(uid_hbm, iid_hbm, ut_hbm, it_hbm, w_hbm, b_hbm, out_hbm,
               uidx_v, iidx_v, uq_v, iq_v, ublk_v, iblk_v, w_v, b_v, out_v,
               usem, isem):
    wid = lax.axis_index("s") * NC + lax.axis_index("c")
    base = wid * BPW

    for c in range(NCHUNK):
        pltpu.sync_copy(uid_hbm.at[pl.ds(base + c * CHUNK, CHUNK)],
                        uidx_v.at[c])
        pltpu.sync_copy(iid_hbm.at[pl.ds(base + c * CHUNK, CHUNK)],
                        iidx_v.at[c])
    pltpu.sync_copy(w_hbm, w_v)
    pltpu.sync_copy(b_hbm, b_v)

    for c in range(NCHUNK):
        for t in range(CHUNK // L):
            uq_v[c, pl.ds(t * L, L)] = (
                lax.shift_right_logical(uidx_v[c, pl.ds(t * L, L)], 1))
            iq_v[c, pl.ds(t * L, L)] = (
                lax.shift_right_logical(iidx_v[c, pl.ds(t * L, L)], 1))

    wvecs = [w_v[pl.ds(j * L, L)] for j in range(D // L)]
    bvec = b_v[...]
    liota = lax.iota(jnp.int32, L)

    for h in range(2):
        copies = []
        for c in range(HALF // CHUNK):
            copies.append(pltpu.async_copy(
                ut_hbm.at[uq_v.at[h * (HALF // CHUNK) + c]],
                ublk_v.at[pl.ds(c * CHUNK, CHUNK)], usem))
            copies.append(pltpu.async_copy(
                it_hbm.at[iq_v.at[h * (HALF // CHUNK) + c]],
                iblk_v.at[pl.ds(c * CHUNK, CHUNK)], isem))
        for cp in copies:
            cp.wait()

        def group_body(g, carry):
            r0 = g * L
            gpos = h * HALF + r0
            cc = lax.shift_right_logical(gpos, 7)
            oo = lax.bitwise_and(gpos, 127)
            uvec = uidx_v[cc, pl.ds(oo, L)]
            ivec = iidx_v[cc, pl.ds(oo, L)]
            acc = bvec
            for rr in range(L):
                r = r0 + rr
                pu = (uvec[rr] % 2) * D
                pi = (ivec[rr] % 2) * D
                s = None
                for j in range(D // L):
                    t = (ublk_v[r, pl.ds(pu + j * L, L)]
                         * iblk_v[r, pl.ds(pi + j * L, L)] * wvecs[j])
                    s = t if s is None else s + t
                acc = jnp.where(liota == rr, acc + jnp.sum(s), acc)
            out_v[pl.ds(h * HALF + r0, L)] = acc
            return carry

        lax.fori_loop(0, NGROUP, group_body, 0, unroll=False)

    pltpu.sync_copy(out_v, out_hbm.at[pl.ds(base, BPW)])


def kernel(user_ids, item_ids, user_table, item_table, fc_w, fc_b):
    ut2 = user_table.reshape(user_table.shape[0] // 2, DP)
    it2 = item_table.reshape(item_table.shape[0] // 2, DP)
    w = fc_w.reshape(D)
    b = jnp.broadcast_to(fc_b.reshape(1), (L,))
    return _sc_kernel(user_ids, item_ids, ut2, it2, w, b)


# speedup vs baseline: 1.0005x; 1.0005x over previous
import functools

import jax
import jax.numpy as jnp
from jax import lax
from jax.experimental import pallas as pl
from jax.experimental.pallas import tpu as pltpu
from jax.experimental.pallas import tpu_sc as plsc

B = 16384
D = 64
DP = 2 * D
L = 16
NC = 2
NS = 16
NW = NC * NS
BPW = B // NW
CHUNK = 128
NCHUNK = BPW // CHUNK
HALF = 256
NGROUP = HALF // L

_mesh = plsc.VectorSubcoreMesh(core_axis_name="c", subcore_axis_name="s")
_params = pltpu.CompilerParams(needs_layout_passes=False)


def _stage_indices(id_hbm, base, idx_v, q_v):
    for c in range(NCHUNK):
        pltpu.sync_copy(id_hbm.at[pl.ds(base + c * CHUNK, CHUNK)],
                        idx_v.at[c])
    for c in range(NCHUNK):
        for t in range(CHUNK // L):
            q_v[c, pl.ds(t * L, L)] = (
                lax.shift_right_logical(idx_v[c, pl.ds(t * L, L)], 1))


@functools.partial(
    pl.kernel,
    mesh=_mesh,
    compiler_params=_params,
    out_type=jax.ShapeDtypeStruct((B // 2, DP), jnp.float32),
    scratch_types=[
        pltpu.VMEM((NCHUNK, CHUNK), jnp.int32),
        pltpu.VMEM((NCHUNK, CHUNK), jnp.int32),
        pltpu.VMEM((HALF, DP), jnp.float32),
        pltpu.VMEM((HALF // 2, DP), jnp.float32),
        pltpu.VMEM((D,), jnp.float32),
        pltpu.SemaphoreType.DMA,
    ],
)
def _user_kernel(uid_hbm, ut_hbm, w_hbm, uw_hbm,
                 uidx_v, uq_v, ublk_v, urow_v, w_v, usem):
    wid = lax.axis_index("s") * NC + lax.axis_index("c")
    base = wid * BPW

    _stage_indices(uid_hbm, base, uidx_v, uq_v)
    pltpu.sync_copy(w_hbm, w_v)
    wvecs = [w_v[pl.ds(j * L, L)] for j in range(D // L)]

    for h in range(2):
        copies = []
        for c in range(HALF // CHUNK):
            copies.append(pltpu.async_copy(
                ut_hbm.at[uq_v.at[h * (HALF // CHUNK) + c]],
                ublk_v.at[pl.ds(c * CHUNK, CHUNK)], usem))
        for cp in copies:
            cp.wait()

        def row_group(g, carry):
            r0 = g * L
            gpos = h * HALF + r0
            cc = lax.shift_right_logical(gpos, 7)
            oo = lax.bitwise_and(gpos, 127)
            uvec = uidx_v[cc, pl.ds(oo, L)]
            for rr in range(L):
                r = r0 + rr
                pu = (uvec[rr] % 2) * D
                rq = lax.shift_right_logical(r, 1)
                ro = (rr % 2) * D
                for j in range(D // L):
                    urow_v[rq, pl.ds(ro + j * L, L)] = (
                        ublk_v[r, pl.ds(pu + j * L, L)] * wvecs[j])
            return carry

        lax.fori_loop(0, NGROUP, row_group, 0, unroll=False)

        pltpu.sync_copy(
            urow_v,
            uw_hbm.at[pl.ds(
                pl.multiple_of((base + h * HALF) // 2, HALF // 2),
                HALF // 2)])


@functools.partial(
    pl.kernel,
    mesh=_mesh,
    compiler_params=_params,
    out_type=jax.ShapeDtypeStruct((B,), jnp.float32),
    scratch_types=[
        pltpu.VMEM((NCHUNK, CHUNK), jnp.int32),
        pltpu.VMEM((NCHUNK, CHUNK), jnp.int32),
        pltpu.VMEM((HALF, DP), jnp.float32),
        pltpu.VMEM((HALF // 2, DP), jnp.float32),
        pltpu.VMEM((L,), jnp.float32),
        pltpu.VMEM((BPW,), jnp.float32),
        pltpu.SemaphoreType.DMA,
    ],
)
def _item_kernel(iid_hbm, it_hbm, uw_hbm, b_hbm, out_hbm,
                 iidx_v, iq_v, iblk_v, urow_v, b_v, out_v, isem):
    wid = lax.axis_index("s") * NC + lax.axis_index("c")
    base = wid * BPW

    _stage_indices(iid_hbm, base, iidx_v, iq_v)
    pltpu.sync_copy(b_hbm, b_v)
    bvec = b_v[...]
    liota = lax.iota(jnp.int32, L)

    for h in range(2):
        copies = [pltpu.async_copy(
            it_hbm.at[iq_v.at[h * (HALF // CHUNK) + c]],
            iblk_v.at[pl.ds(c * CHUNK, CHUNK)], isem)
            for c in range(HALF // CHUNK)]
        pltpu.sync_copy(
            uw_hbm.at[pl.ds(
                pl.multiple_of((base + h * HALF) // 2, HALF // 2),
                HALF // 2)], urow_v)
        for cp in copies:
            cp.wait()

        def group_body(g, carry):
            r0 = g * L
            gpos = h * HALF + r0
            cc = lax.shift_right_logical(gpos, 7)
            oo = lax.bitwise_and(gpos, 127)
            ivec = iidx_v[cc, pl.ds(oo, L)]
            acc = bvec
            for rr in range(L):
                r = r0 + rr
                pi = (ivec[rr] % 2) * D
                rq = r // 2
                ro = (r % 2) * D
                s = None
                for j in range(D // L):
                    t = (urow_v[rq, pl.ds(ro + j * L, L)]
                         * iblk_v[r, pl.ds(pi + j * L, L)])
                    s = t if s is None else s + t
                acc = jnp.where(liota == rr, acc + jnp.sum(s), acc)
            out_v[pl.ds(h * HALF + r0, L)] = acc
            return carry

        lax.fori_loop(0, NGROUP, group_body, 0, unroll=False)

    pltpu.sync_copy(out_v, out_hbm.at[pl.ds(base, BPW)])


def kernel(user_ids, item_ids, user_table, item_table, fc_w, fc_b):
    ut2 = user_table.reshape(user_table.shape[0] // 2, DP)
    it2 = item_table.reshape(item_table.shape[0] // 2, DP)
    w = fc_w.reshape(D)
    b = jnp.broadcast_to(fc_b.reshape(1), (L,))
    uw = _user_kernel(user_ids, ut2, w)
    return _item_kernel(item_ids, it2, uw, b)
